# Initial kernel scaffold; baseline (speedup 1.0000x reference)
import jax
import jax.numpy as jnp
from jax.experimental import pallas as pl


def kernel(idxs, boxes):
    raise NotImplementedError("write your pallas kernel here")


# speedup vs baseline: 1.9409x; 1.9409x over previous
import functools

import jax
import jax.numpy as jnp
from jax import lax
from jax.experimental import pallas as pl
from jax.experimental.pallas import tpu as pltpu
from jax.experimental.pallas import tpu_sc as plsc

N_ENT = 1000000
DIM = 32
BATCH = 16384
INT_TEMP = 0.01
VOL_TEMP = 1.0

NC, NS = 2, 16
NW = NC * NS
ROWS = 2 * BATCH
R_PER_W = ROWS // NW
CH = 128
NCH = R_PER_W // CH

@functools.lru_cache(maxsize=None)
def _get_gather():
    mesh = plsc.VectorSubcoreMesh(
        core_axis_name="c", subcore_axis_name="s", num_cores=NC, num_subcores=NS
    )

    @functools.partial(
        pl.kernel,
        mesh=mesh,
        out_type=jax.ShapeDtypeStruct((ROWS, 2 * DIM), jnp.float32),
        scratch_types=[
            pltpu.VMEM((NCH, CH), jnp.int32),
            pltpu.VMEM((R_PER_W, 2 * DIM), jnp.float32),
            pltpu.SemaphoreType.DMA,
        ],
        compiler_params=pltpu.CompilerParams(use_tc_tiling_on_sc=False),
    )
    def _gather(table_hbm, idx_hbm, out_hbm, idx_v, rows_v, sem):
        wid = lax.axis_index("s") * NC + lax.axis_index("c")
        base = wid * R_PER_W
        pltpu.sync_copy(idx_hbm.at[wid], idx_v)
        copies = []
        for j in range(NCH):
            copies.append(
                pltpu.async_copy(
                    table_hbm.at[idx_v.at[j]],
                    rows_v.at[pl.ds(j * CH, CH)],
                    sem,
                )
            )
        for c in copies:
            c.wait()
        pltpu.sync_copy(rows_v, out_hbm.at[pl.ds(base, R_PER_W)])

    return _gather


_TC_ROWS = 2048


def _energy_body(g_ref, out_ref):
    g = g_ref[...]
    z0 = g[:, 0:DIM]
    nz0 = g[:, DIM:2 * DIM]
    z1 = g[:, 2 * DIM:3 * DIM]
    nz1 = g[:, 3 * DIM:4 * DIM]

    def gumbel_lse(a, b):
        m = jnp.maximum(a, b)
        lo = jnp.minimum(a, b)
        return m + INT_TEMP * jnp.log1p(jnp.exp((lo - m) / INT_TEMP))

    inter_z = gumbel_lse(z0, z1)
    inter_nz = gumbel_lse(nz0, nz1)
    side0 = -(inter_z + inter_nz)
    side1 = -(z1 + nz1)

    def log_vol_terms(s):
        s = s / VOL_TEMP
        sp = jnp.maximum(s, 0.0) + jnp.log1p(jnp.exp(-jnp.abs(s)))
        return jnp.log(VOL_TEMP * sp + 1e-23)

    d = jnp.sum(log_vol_terms(side0) - log_vol_terms(side1), axis=1)
    out_ref[...] = d.reshape(_TC_ROWS // 128, 128)


_energy = pl.pallas_call(
    _energy_body,
    grid=(BATCH // _TC_ROWS,),
    in_specs=[pl.BlockSpec((_TC_ROWS, 4 * DIM), lambda i: (i, 0))],
    out_specs=pl.BlockSpec((_TC_ROWS // 128, 128), lambda i: (i, 0)),
    out_shape=jax.ShapeDtypeStruct((BATCH // 128, 128), jnp.float32),
)


def kernel(idxs, boxes):
    idx3 = idxs.reshape(NW, NCH, CH)
    table = boxes.reshape(N_ENT, 2 * DIM)
    g = _get_gather()(table, idx3)
    out = _energy(g.reshape(BATCH, 4 * DIM))
    return out.reshape(BATCH)
